# baseline (speedup 1.0000x reference)
import jax
import jax.numpy as jnp
from jax.experimental import pallas as pl
from jax.experimental.pallas import tpu as pltpu

_BLOCK_ROWS = 512


def _scale_body(w_ref, x_ref, o_ref):
    f = w_ref[0, 0]
    o_ref[...] = x_ref[...] * (f * f)


def kernel(x, multiple_factor_weight):
    b, r, c = x.shape
    x2d = x.reshape(b * r, c)
    n_rows = b * r
    grid = (n_rows // _BLOCK_ROWS,)
    out = pl.pallas_call(
        _scale_body,
        grid=grid,
        in_specs=[
            pl.BlockSpec(memory_space=pltpu.MemorySpace.SMEM),
            pl.BlockSpec((_BLOCK_ROWS, c), lambda i: (i, 0)),
        ],
        out_specs=pl.BlockSpec((_BLOCK_ROWS, c), lambda i: (i, 0)),
        out_shape=jax.ShapeDtypeStruct((n_rows, c), x.dtype),
        compiler_params=pltpu.CompilerParams(
            dimension_semantics=("arbitrary",),
        ),
    )(multiple_factor_weight, x2d)
    return out.reshape(b, r, c)
